# Initial kernel scaffold; baseline (speedup 1.0000x reference)
import jax
import jax.numpy as jnp
from jax.experimental import pallas as pl


def kernel(params, z, edge_index, bond_type, batch):
    raise NotImplementedError("write your pallas kernel here")


# speedup vs baseline: 7.5404x; 7.5404x over previous
import functools

import jax
import jax.numpy as jnp
from jax import lax
from jax.experimental import pallas as pl
from jax.experimental.pallas import tpu as pltpu
from jax.experimental.pallas import tpu_sc as plsc

N = 10000
E = 160000
T = 16
C = 32
NZ = 100
NPAD = 10240
TRASH = 10200
NC, NS = 2, 16
NW = NC * NS
EPT = 5120
CH = 128
NCH = EPT // CH
ROWS_PER_TILE = NPAD // NS


def _emb_body(z_ref, emb_ref, w_ref, b_ref, out_ref):
    zc = z_ref[...]
    oh = (lax.broadcasted_iota(jnp.int32, (N, NZ), 1) == zc).astype(jnp.float32)
    e1 = jax.nn.relu(jnp.dot(oh, emb_ref[...], preferred_element_type=jnp.float32))
    x0 = jax.nn.relu(jnp.dot(e1, w_ref[...], preferred_element_type=jnp.float32)
                     + b_ref[...])
    out_ref[...] = x0


def _node_embed(z, emb, w, b):
    return pl.pallas_call(
        _emb_body,
        out_shape=jax.ShapeDtypeStruct((N, C), jnp.float32),
    )(z.reshape(N, 1).astype(jnp.int32), emb, w, b.reshape(1, C))


def _wall_body(be_ref, w1_ref, b1_ref, w2_ref, b2_ref, out_ref):
    h = jax.nn.relu(jnp.dot(be_ref[...], w1_ref[...],
                            preferred_element_type=jnp.float32) + b1_ref[...])
    out_ref[...] = (jnp.dot(h, w2_ref[...], preferred_element_type=jnp.float32)
                    + b2_ref[...])


def _edge_weights(bond_emb, w1, b1, w2, b2):
    return pl.pallas_call(
        _wall_body,
        out_shape=jax.ShapeDtypeStruct((T, C * C), jnp.float32),
    )(bond_emb, w1, b1.reshape(1, C), w2, b2.reshape(1, C * C))


def _z_body(x_ref, wt_ref, out_ref):
    wt = wt_ref[...].reshape(C, C)
    zt = jnp.dot(x_ref[...], wt, preferred_element_type=jnp.float32)
    out_ref[...] = zt.reshape(1, N, C)


def _z_all_types(x, wall3):
    return pl.pallas_call(
        _z_body,
        grid=(T,),
        in_specs=[
            pl.BlockSpec((N, C), lambda t: (0, 0)),
            pl.BlockSpec((1, C, C), lambda t: (t, 0, 0)),
        ],
        out_specs=pl.BlockSpec((1, N, C), lambda t: (t, 0, 0)),
        out_shape=jax.ShapeDtypeStruct((T, N, C), jnp.float32),
    )(x, wall3)


def _update_body(agg_ref, cm_ref, x_ref, root_ref, bias_ref, out_ref, cnt_ref):
    cnt = (cm_ref[0] + cm_ref[1])[:, 0:1]
    agg = agg_ref[0] + agg_ref[1]
    mean = agg / jnp.clip(cnt, 1.0)
    out_ref[...] = jax.nn.relu(
        mean + jnp.dot(x_ref[...], root_ref[...],
                       preferred_element_type=jnp.float32) + bias_ref[...])
    cnt_ref[...] = cnt


_RB = 2000


def _node_update1(agg, cntmat, x, root, bias):
    return pl.pallas_call(
        _update_body,
        grid=(N // _RB,),
        in_specs=[
            pl.BlockSpec((2, _RB, C), lambda i: (0, i, 0)),
            pl.BlockSpec((2, _RB, 16), lambda i: (0, i, 0)),
            pl.BlockSpec((_RB, C), lambda i: (i, 0)),
            pl.BlockSpec((C, C), lambda i: (0, 0)),
            pl.BlockSpec((1, C), lambda i: (0, 0)),
        ],
        out_specs=(pl.BlockSpec((_RB, C), lambda i: (i, 0)),
                   pl.BlockSpec((_RB, 1), lambda i: (i, 0))),
        out_shape=(jax.ShapeDtypeStruct((N, C), jnp.float32),
                   jax.ShapeDtypeStruct((N, 1), jnp.float32)),
    )(agg, cntmat, x, root, bias.reshape(1, C))


def _update2_body(agg_ref, cnt_ref, x_ref, root_ref, bias_ref, out_ref):
    agg = agg_ref[0] + agg_ref[1]
    mean = agg / jnp.clip(cnt_ref[...], 1.0)
    out_ref[...] = jax.nn.relu(
        mean + jnp.dot(x_ref[...], root_ref[...],
                       preferred_element_type=jnp.float32) + bias_ref[...])


def _node_update2(agg, cnt, x, root, bias):
    return pl.pallas_call(
        _update2_body,
        grid=(N // _RB,),
        in_specs=[
            pl.BlockSpec((2, _RB, C), lambda i: (0, i, 0)),
            pl.BlockSpec((_RB, 1), lambda i: (i, 0)),
            pl.BlockSpec((_RB, C), lambda i: (i, 0)),
            pl.BlockSpec((C, C), lambda i: (0, 0)),
            pl.BlockSpec((1, C), lambda i: (0, 0)),
        ],
        out_specs=pl.BlockSpec((_RB, C), lambda i: (i, 0)),
        out_shape=jax.ShapeDtypeStruct((N, C), jnp.float32),
    )(agg, cnt, x, root, bias.reshape(1, C))


def _final_body(x_ref, batch_ref,
                wih_ref, whh_ref, bih_ref, bhh_ref,
                r1w_ref, r1b_ref, r2w_ref, r2b_ref, out_ref):
    x = x_ref[...]
    B = 64
    oh = (lax.broadcasted_iota(jnp.int32, (N, B), 1)
          == batch_ref[...]).astype(jnp.float32)

    def segsum(v):
        return lax.dot_general(oh, v, (((0,), (0,)), ((), ())),
                               preferred_element_type=jnp.float32,
                               precision=lax.Precision.HIGHEST)

    wih = wih_ref[...]
    whh = whh_ref[...]
    q_star = jnp.zeros((B, 2 * C), jnp.float32)
    hs = jnp.zeros((B, C), jnp.float32)
    cs = jnp.zeros((B, C), jnp.float32)
    for _ in range(3):
        gates = (lax.dot_general(q_star, wih, (((1,), (1,)), ((), ())),
                                 preferred_element_type=jnp.float32)
                 + bih_ref[...]
                 + lax.dot_general(hs, whh, (((1,), (1,)), ((), ())),
                                   preferred_element_type=jnp.float32)
                 + bhh_ref[...])
        i = jax.nn.sigmoid(gates[:, 0:C])
        f = jax.nn.sigmoid(gates[:, C:2 * C])
        g = jnp.tanh(gates[:, 2 * C:3 * C])
        o = jax.nn.sigmoid(gates[:, 3 * C:4 * C])
        cs = f * cs + i * g
        hs = o * jnp.tanh(cs)
        q = hs
        qb = jnp.dot(oh, q, preferred_element_type=jnp.float32,
                     precision=lax.Precision.HIGHEST)
        e = jnp.sum(x * qb, axis=-1, keepdims=True)
        masked = jnp.where(oh > 0.0, e, -jnp.inf)
        emax = jnp.max(masked, axis=0, keepdims=True)
        emax = jnp.where(jnp.isfinite(emax), emax, 0.0)
        eb = jnp.dot(oh, emax.reshape(B, 1),
                     preferred_element_type=jnp.float32,
                     precision=lax.Precision.HIGHEST)
        ee = jnp.exp(e - eb)
        denom = segsum(ee)
        a = ee / (jnp.dot(oh, denom, preferred_element_type=jnp.float32,
                          precision=lax.Precision.HIGHEST)
                  + 1e-16)
        r = segsum(a * x)
        q_star = jnp.concatenate([q, r], axis=1)
    out = jax.nn.relu(jnp.dot(q_star, r1w_ref[...],
                              preferred_element_type=jnp.float32) + r1b_ref[...])
    out = jnp.dot(out, r2w_ref[...],
                  preferred_element_type=jnp.float32) + r2b_ref[...]
    out_ref[...] = out


def _set2set(x, batch, p):
    return pl.pallas_call(
        _final_body,
        out_shape=jax.ShapeDtypeStruct((64, 1), jnp.float32),
    )(x, batch.reshape(N, 1).astype(jnp.int32),
      p['lstm_Wih'], p['lstm_Whh'], p['lstm_bih'].reshape(1, 4 * C),
      p['lstm_bhh'].reshape(1, 4 * C), p['red1_W'], p['red1_b'].reshape(1, C),
      p['red2_W'], p['red2_b'].reshape(1, 1))


def _sc_body(do_cnt, *refs):
    if do_cnt:
        (z_hbm, gidx_hbm, sidx_hbm, agg_out, cnt_out,
         gidx_v, sidx_v, buf, zb, zb16, ones_v, agg_sh, cnt_sh, sem) = refs
    else:
        (z_hbm, gidx_hbm, sidx_hbm, agg_out,
         gidx_v, sidx_v, buf, zb, agg_sh, sem) = refs
    cid = lax.axis_index("c")
    sid = lax.axis_index("s")
    wid = cid * NS + sid

    pltpu.sync_copy(gidx_hbm.at[wid], gidx_v)
    pltpu.sync_copy(sidx_hbm.at[wid], sidx_v)

    zf = jnp.zeros((16,), jnp.float32)

    def zrow(i, _):
        zb[i, 0:16] = zf
        zb[i, 16:32] = zf
        if do_cnt:
            zb16[i, 0:16] = zf
            ones_v[i, 0:16] = zf + 1.0
        return 0

    lax.fori_loop(0, CH, zrow, 0)
    base = sid * ROWS_PER_TILE
    for j in range(ROWS_PER_TILE // CH):
        pltpu.sync_copy(zb, agg_sh.at[pl.ds(base + j * CH, CH)])
        if do_cnt:
            pltpu.sync_copy(zb16, cnt_sh.at[pl.ds(base + j * CH, CH)])
    plsc.subcore_barrier()

    def chunk(c, _):
        pltpu.async_copy(z_hbm.at[gidx_v.at[c]], buf, sem).wait()
        pltpu.sync_copy(buf, agg_sh.at[sidx_v.at[c]], add=True)
        if do_cnt:
            pltpu.sync_copy(ones_v, cnt_sh.at[sidx_v.at[c]], add=True)
        return 0

    lax.fori_loop(0, NCH, chunk, 0)
    plsc.subcore_barrier()

    pltpu.sync_copy(agg_sh.at[pl.ds(base, ROWS_PER_TILE)],
                    agg_out.at[cid, pl.ds(base, ROWS_PER_TILE)])
    if do_cnt:
        pltpu.sync_copy(cnt_sh.at[pl.ds(base, ROWS_PER_TILE)],
                        cnt_out.at[cid, pl.ds(base, ROWS_PER_TILE)])


def _sc_gather_scatter(z2d, gidx, sidx, do_cnt):
    out_type = [jax.ShapeDtypeStruct((NC, NPAD, C), jnp.float32)]
    scratch = [
        pltpu.VMEM((NCH, CH), jnp.int32),
        pltpu.VMEM((NCH, CH), jnp.int32),
        pltpu.VMEM((CH, C), jnp.float32),
        pltpu.VMEM((CH, C), jnp.float32),
    ]
    if do_cnt:
        out_type.append(jax.ShapeDtypeStruct((NC, NPAD, 16), jnp.float32))
        scratch += [
            pltpu.VMEM((CH, 16), jnp.float32),
            pltpu.VMEM((CH, 16), jnp.float32),
        ]
    scratch.append(pltpu.VMEM_SHARED((NPAD, C), jnp.float32))
    if do_cnt:
        scratch.append(pltpu.VMEM_SHARED((NPAD, 16), jnp.float32))
    scratch.append(pltpu.SemaphoreType.DMA)
    fn = pl.kernel(
        functools.partial(_sc_body, do_cnt),
        out_type=tuple(out_type),
        mesh=plsc.VectorSubcoreMesh(core_axis_name="c", subcore_axis_name="s",
                                    num_cores=NC, num_subcores=NS),
        scratch_types=tuple(scratch),
        compiler_params=pltpu.CompilerParams(use_tc_tiling_on_sc=False),
    )
    return fn(z2d, gidx, sidx)


def kernel(params, z, edge_index, bond_type, batch):
    p = params
    src = edge_index[0].astype(jnp.int32)
    dst = edge_index[1].astype(jnp.int32)
    bt = bond_type.astype(jnp.int32)
    npad = NW * EPT - E
    gidx = jnp.concatenate([bt * N + src, jnp.zeros((npad,), jnp.int32)])
    sidx = jnp.concatenate([dst, jnp.full((npad,), TRASH, jnp.int32)])
    gidx = gidx.reshape(NW, NCH, CH)
    sidx = sidx.reshape(NW, NCH, CH)

    x = _node_embed(z, p['node_emb'], p['node_lin_W'], p['node_lin_b'])

    lp = p['layers'][0]
    wall = _edge_weights(p['bond_emb'], lp['nn1_W'], lp['nn1_b'],
                         lp['nn2_W'], lp['nn2_b'])
    z1 = _z_all_types(x, wall.reshape(T, C, C))
    agg1, cntmat = _sc_gather_scatter(z1.reshape(T * N, C), gidx, sidx, True)
    x, cnt = _node_update1(agg1, cntmat, x, lp['root_W'], lp['bias'])

    lp = p['layers'][1]
    wall = _edge_weights(p['bond_emb'], lp['nn1_W'], lp['nn1_b'],
                         lp['nn2_W'], lp['nn2_b'])
    z2 = _z_all_types(x, wall.reshape(T, C, C))
    (agg2,) = _sc_gather_scatter(z2.reshape(T * N, C), gidx, sidx, False)
    x = _node_update2(agg2, cnt, x, lp['root_W'], lp['bias'])
    return _set2set(x, batch, p)
